# baseline (speedup 1.0000x reference)
import functools

import jax
import jax.numpy as jnp
import numpy as np
from jax.experimental import pallas as pl
from jax.experimental.pallas import tpu as pltpu

_B, _N, _M, _D = 8, 256, 16, 256
_LOG2E = 1.4426950408889634
_LN2 = 0.6931471805599453

_ASEL = ((np.arange(_N)[:, None] == (np.arange(_N * _M)[None, :] // _M))
         .astype(np.float32))
_DSEL = ((np.arange(8)[:, None] == ((np.arange(_N * _M) % 128) // _M)[None, :])
         .astype(np.float32))


def _one_batch(e4, alpha, dsel):
    n, m, d = _N, _M, _D
    nm = n * m

    e = e4.reshape(nm, d)
    e_bf = e.astype(jnp.bfloat16)
    e_t = jnp.transpose(e_bf)

    csum = jnp.sum(e4, axis=1)
    cmat = (csum * (alpha * (1.0 / m))).astype(jnp.bfloat16)

    d2t = jax.lax.dot_general(
        cmat, e_t, (((1,), (0,)), ((), ())),
        preferred_element_type=jnp.float32)

    esq_t = e_t * e_t
    trow = jax.lax.dot_general(
        jnp.ones((8, d), jnp.bfloat16), esq_t, (((1,), (0,)), ((), ())),
        preferred_element_type=jnp.float32)[0:1]

    dg = jnp.concatenate(
        [d2t[8 * c:8 * c + 8, 128 * c:128 * c + 128] for c in range(32)],
        axis=1)
    diag = jnp.sum(dg * dsel, axis=0, keepdims=True)

    self2 = diag * (m / (m - 1.0)) - trow * (alpha / (m - 1.0))
    dgmod = dg + dsel * (self2 - dg)
    bands = []
    for c in range(32):
        band = d2t[8 * c:8 * c + 8, :]
        parts = []
        if c > 0:
            parts.append(band[:, :128 * c])
        parts.append(dgmod[:, 128 * c:128 * c + 128])
        if c < 31:
            parts.append(band[:, 128 * c + 128:])
        bands.append(jnp.concatenate(parts, axis=1) if len(parts) > 1
                     else parts[0])
    d2m = jnp.concatenate(bands, axis=0)

    mx = jnp.max(d2m, axis=0, keepdims=True)
    ssum = jnp.sum(jnp.exp2(d2m - mx), axis=0, keepdims=True)
    lse = mx + jnp.log2(ssum)
    return jnp.sum(lse - self2) * _LN2


def _loss_kernel(w_ref, dsel_ref, ea_ref, eb_ref, o_ref):
    alpha = w_ref[0] * _LOG2E
    dsel = dsel_ref[...]
    partial = 0.0
    for g in range(ea_ref.shape[0]):
        e4 = jnp.concatenate([ea_ref[g], eb_ref[g]], axis=0)
        partial += _one_batch(e4, alpha, dsel)

    i = pl.program_id(0)

    @pl.when(i == 0)
    def _():
        o_ref[0, 0] = partial

    @pl.when(i != 0)
    def _():
        o_ref[0, 0] += partial


@functools.partial(jax.jit, static_argnames=())
def kernel(embeddings, w, b):
    del b
    bsz, n, m, d = embeddings.shape
    w1 = jnp.reshape(w.astype(jnp.float32), (1,))
    partials = pl.pallas_call(
        _loss_kernel,
        grid=(bsz,),
        in_specs=[
            pl.BlockSpec(memory_space=pltpu.SMEM),
            pl.BlockSpec((8, n * m), lambda i: (0, 0)),
            pl.BlockSpec((1, n // 2, m, d), lambda i: (i, 0, 0, 0)),
            pl.BlockSpec((1, n // 2, m, d), lambda i: (i, 1, 0, 0)),
        ],
        out_specs=pl.BlockSpec(memory_space=pltpu.SMEM),
        out_shape=jax.ShapeDtypeStruct((1, 1), jnp.float32),
        compiler_params=pltpu.CompilerParams(
            dimension_semantics=("parallel",),
            vmem_limit_bytes=100 * 1024 * 1024,
        ),
    )(w1, jnp.asarray(_DSEL), embeddings, embeddings)
    return partials[0, 0]


# speedup vs baseline: 1.0186x; 1.0186x over previous
import functools

import jax
import jax.numpy as jnp
import numpy as np
from jax.experimental import pallas as pl
from jax.experimental.pallas import tpu as pltpu

_B, _N, _M, _D = 8, 256, 16, 256
_LOG2E = 1.4426950408889634
_LN2 = 0.6931471805599453

_DSEL = ((np.arange(8)[:, None] == ((np.arange(_N * _M) % 128) // _M)[None, :])
         .astype(np.float32))


def _one_batch(e4, alpha, dsel):
    n, m, d = _N, _M, _D
    nm = n * m

    e = e4.reshape(nm, d)
    e_bf = e.astype(jnp.bfloat16)
    e_t = jnp.transpose(e_bf)

    csum = jnp.sum(e4, axis=1)
    cmat = (csum * (alpha * (1.0 / m))).astype(jnp.bfloat16)

    d2t = jax.lax.dot_general(
        cmat, e_t, (((1,), (0,)), ((), ())),
        preferred_element_type=jnp.float32)

    esq_t = e_t * e_t
    trow = jax.lax.dot_general(
        jnp.ones((8, d), jnp.bfloat16), esq_t, (((1,), (0,)), ((), ())),
        preferred_element_type=jnp.float32)[0:1]

    dg = jnp.concatenate(
        [d2t[8 * c:8 * c + 8, 128 * c:128 * c + 128] for c in range(32)],
        axis=1)
    diag = jnp.sum(dg * dsel, axis=0, keepdims=True)

    self2 = diag * (m / (m - 1.0)) - trow * (alpha / (m - 1.0))
    dgmod = dg + dsel * (self2 - dg)
    bands = []
    for c in range(32):
        band = d2t[8 * c:8 * c + 8, :]
        parts = []
        if c > 0:
            parts.append(band[:, :128 * c])
        parts.append(dgmod[:, 128 * c:128 * c + 128])
        if c < 31:
            parts.append(band[:, 128 * c + 128:])
        bands.append(jnp.concatenate(parts, axis=1) if len(parts) > 1
                     else parts[0])
    d2m = jnp.concatenate(bands, axis=0)

    mx = jnp.max(d2m, axis=0, keepdims=True)
    ssum = jnp.sum(jnp.exp2(d2m - mx), axis=0, keepdims=True)
    lse = mx + jnp.log2(ssum)
    return jnp.sum(lse - self2) * _LN2


def _loss_kernel(w_ref, dsel_ref, ea_ref, eb_ref, o_ref):
    alpha = w_ref[0] * _LOG2E
    dsel = dsel_ref[...]
    partial = 0.0
    for g in range(2):
        e4 = jnp.concatenate([ea_ref[g], eb_ref[g]], axis=0)
        partial += _one_batch(e4, alpha, dsel)

    i = pl.program_id(0)

    @pl.when(i == 0)
    def _():
        o_ref[0, 0] = partial

    @pl.when(i != 0)
    def _():
        o_ref[0, 0] += partial


@functools.partial(jax.jit, static_argnames=())
def kernel(embeddings, w, b):
    del b
    bsz, n, m, d = embeddings.shape
    w1 = jnp.reshape(w.astype(jnp.float32), (1,))
    partials = pl.pallas_call(
        _loss_kernel,
        grid=(bsz // 2,),
        in_specs=[
            pl.BlockSpec(memory_space=pltpu.SMEM),
            pl.BlockSpec((8, n * m), lambda i: (0, 0)),
            pl.BlockSpec((2, n // 2, m, d), lambda i: (i, 0, 0, 0)),
            pl.BlockSpec((2, n // 2, m, d), lambda i: (i, 1, 0, 0)),
        ],
        out_specs=pl.BlockSpec(memory_space=pltpu.SMEM),
        out_shape=jax.ShapeDtypeStruct((1, 1), jnp.float32),
        compiler_params=pltpu.CompilerParams(
            dimension_semantics=("parallel",),
            vmem_limit_bytes=100 * 1024 * 1024,
        ),
    )(w1, jnp.asarray(_DSEL), embeddings, embeddings)
    return partials[0, 0]
